# speedup vs baseline: 1.2586x; 1.2586x over previous
import jax
import jax.numpy as jnp
from jax.experimental import pallas as pl

N_CLASS = 21
N_FG = N_CLASS - 1
N_ROI = 1000
N_PAD = 1024
IMG_H, IMG_W = 600, 800
SCORE_LOW = 0.05
NMS_THRESH = 0.3
J_CHUNK = 256
NEG = -1e30


def _softmax_prob(score_rows, own):
    rm = score_rows[0]
    for c in range(1, N_CLASS):
        rm = jnp.maximum(rm, score_rows[c])
    rs = jnp.exp(score_rows[0] - rm)
    for c in range(1, N_CLASS):
        rs = rs + jnp.exp(score_rows[c] - rm)
    return jnp.exp(own - rm) / rs


def _decode(sy1, sx1, sy2, sx2, dy, dx, dh, dw):
    src_h = sy2 - sy1
    src_w = sx2 - sx1
    src_cy = sy1 + 0.5 * src_h
    src_cx = sx1 + 0.5 * src_w
    cy = dy * src_h + src_cy
    cx = dx * src_w + src_cx
    h = jnp.exp(dh) * src_h
    w = jnp.exp(dw) * src_w
    by1 = jnp.clip(cy - 0.5 * h, 0.0, float(IMG_H))
    bx1 = jnp.clip(cx - 0.5 * w, 0.0, float(IMG_W))
    by2 = jnp.clip(cy + 0.5 * h, 0.0, float(IMG_H))
    bx2 = jnp.clip(cx + 0.5 * w, 0.0, float(IMG_W))
    return by1, bx1, by2, bx2


def _nms_kernel(locrow_ref, auxrow_ref, loccol_ref, auxcol_ref, out_ref):
    lr = locrow_ref[0]
    dy_r, dx_r, dh_r, dw_r = (lr[k : k + 1, :] for k in range(4))
    s_raw_r = lr[4:5, :]
    ar = auxrow_ref[...]
    sy1_r, sx1_r, sy2_r, sx2_r = (ar[k : k + 1, :] for k in range(4))
    score_rows_r = [ar[4 + c : 5 + c, :] for c in range(N_CLASS)]

    prob_r = _softmax_prob(score_rows_r, s_raw_r)
    s_r = jnp.where(prob_r > SCORE_LOW, prob_r, 0.0)
    by1_r, bx1_r, by2_r, bx2_r = _decode(
        sy1_r, sx1_r, sy2_r, sx2_r, dy_r, dx_r, dh_r, dw_r
    )
    area_r = jnp.maximum(by2_r - by1_r, 0.0) * jnp.maximum(bx2_r - bx1_r, 0.0)
    i_idx = jax.lax.broadcasted_iota(jnp.int32, (1, N_PAD), 1)

    max_iou = jnp.zeros((1, N_PAD), jnp.float32)
    for j0 in range(0, N_PAD, J_CHUNK):
        lc = loccol_ref[0, j0 : j0 + J_CHUNK, :]
        dy_c, dx_c, dh_c, dw_c = (lc[:, k : k + 1] for k in range(4))
        s_raw_c = lc[:, 4:5]
        ac = auxcol_ref[j0 : j0 + J_CHUNK, :]
        sy1_c, sx1_c, sy2_c, sx2_c = (ac[:, k : k + 1] for k in range(4))
        score_rows_c = [ac[:, 4 + c : 5 + c] for c in range(N_CLASS)]

        prob_c = _softmax_prob(score_rows_c, s_raw_c)
        s_c = jnp.where(prob_c > SCORE_LOW, prob_c, 0.0)
        by1_c, bx1_c, by2_c, bx2_c = _decode(
            sy1_c, sx1_c, sy2_c, sx2_c, dy_c, dx_c, dh_c, dw_c
        )
        area_c = jnp.maximum(by2_c - by1_c, 0.0) * jnp.maximum(
            bx2_c - bx1_c, 0.0
        )
        j_idx = jax.lax.broadcasted_iota(jnp.int32, (J_CHUNK, 1), 0) + j0

        iy1 = jnp.maximum(by1_c, by1_r)
        ix1 = jnp.maximum(bx1_c, bx1_r)
        iy2 = jnp.minimum(by2_c, by2_r)
        ix2 = jnp.minimum(bx2_c, bx2_r)
        inter = jnp.maximum(iy2 - iy1, 0.0) * jnp.maximum(ix2 - ix1, 0.0)
        union = area_c + area_r - inter
        iou = inter / jnp.maximum(union, 1e-8)
        higher = (s_c > s_r) | ((s_c == s_r) & (j_idx < i_idx))
        masked = jnp.where(higher, iou, 0.0)
        max_iou = jnp.maximum(max_iou, jnp.max(masked, axis=0, keepdims=True))

    keep = (max_iou <= NMS_THRESH) & (s_r > SCORE_LOW)
    out_s = jnp.where(keep, s_r, 0.0)
    out_ref[0] = jnp.concatenate(
        [by1_r, bx1_r, by2_r, bx2_r, out_s, jnp.zeros((3, N_PAD), jnp.float32)],
        axis=0,
    )


@jax.jit
def kernel(rois, roi_cls_loc, roi_score):
    f = jnp.float32
    loc3 = roi_cls_loc.reshape(N_ROI, N_CLASS, 4)

    locrow = jnp.full((N_FG, 8, N_PAD), NEG, f)
    locrow = locrow.at[:, 0:4, :N_ROI].set(loc3[:, 1:, :].transpose(1, 2, 0))
    locrow = locrow.at[:, 4, :N_ROI].set(roi_score[:, 1:].T)
    auxrow = jnp.full((32, N_PAD), NEG, f)
    auxrow = auxrow.at[0:4, :N_ROI].set(rois.T)
    auxrow = auxrow.at[4 : 4 + N_CLASS, :N_ROI].set(roi_score.T)

    loccol = jnp.full((N_FG, N_PAD, 8), NEG, f)
    loccol = loccol.at[:, :N_ROI, 0:4].set(loc3[:, 1:, :].transpose(1, 0, 2))
    loccol = loccol.at[:, :N_ROI, 4].set(roi_score[:, 1:].T)
    auxcol = jnp.full((N_PAD, 32), NEG, f)
    auxcol = auxcol.at[:N_ROI, 0:4].set(rois)
    auxcol = auxcol.at[:N_ROI, 4 : 4 + N_CLASS].set(roi_score)

    out = pl.pallas_call(
        _nms_kernel,
        grid=(N_FG,),
        in_specs=[
            pl.BlockSpec((1, 8, N_PAD), lambda c: (c, 0, 0)),
            pl.BlockSpec((32, N_PAD), lambda c: (0, 0)),
            pl.BlockSpec((1, N_PAD, 8), lambda c: (c, 0, 0)),
            pl.BlockSpec((N_PAD, 32), lambda c: (0, 0)),
        ],
        out_specs=pl.BlockSpec((1, 8, N_PAD), lambda c: (c, 0, 0)),
        out_shape=jax.ShapeDtypeStruct((N_FG, 8, N_PAD), f),
    )(locrow, auxrow, loccol, auxcol)

    bboxes = out[:, 0:4, :N_ROI].transpose(0, 2, 1)
    scores = out[:, 4, :N_ROI]
    return bboxes, scores


# speedup vs baseline: 7.9253x; 6.2969x over previous
import jax
import jax.numpy as jnp
from jax.experimental import pallas as pl
from jax.experimental.pallas import tpu as pltpu

N_CLASS = 21
N_FG = N_CLASS - 1
N_ROI = 1000
N_PAD = 1024
IMG_H, IMG_W = 600, 800
SCORE_LOW = 0.05
NMS_THRESH = 0.3
R_SCALE = NMS_THRESH / (1.0 + NMS_THRESH)
J_CHUNK = 256
NEG = -1e30


def _decode(sy1, sx1, sy2, sx2, dy, dx, dh, dw):
    src_h = sy2 - sy1
    src_w = sx2 - sx1
    src_cy = sy1 + 0.5 * src_h
    src_cx = sx1 + 0.5 * src_w
    cy = dy * src_h + src_cy
    cx = dx * src_w + src_cx
    h = jnp.exp(dh) * src_h
    w = jnp.exp(dw) * src_w
    by1 = jnp.clip(cy - 0.5 * h, 0.0, float(IMG_H))
    bx1 = jnp.clip(cx - 0.5 * w, 0.0, float(IMG_W))
    by2 = jnp.clip(cy + 0.5 * h, 0.0, float(IMG_H))
    bx2 = jnp.clip(cx + 0.5 * w, 0.0, float(IMG_W))
    return by1, bx1, by2, bx2


def _nms_kernel(locrow_ref, auxrow_ref, bbox_ref, score_ref, prob_s):
    c = pl.program_id(0)

    @pl.when(c == 0)
    def _():
        sc = auxrow_ref[4:28, :]
        rm = jnp.max(sc, axis=0, keepdims=True)
        es = jnp.exp(sc - rm)
        rs = jnp.sum(es, axis=0, keepdims=True)
        prob_s[...] = es / rs

    lr = locrow_ref[0]
    dy_r, dx_r, dh_r, dw_r = (lr[k : k + 1, :] for k in range(4))
    ar = auxrow_ref[...]
    sy1_r, sx1_r, sy2_r, sx2_r = (ar[k : k + 1, :] for k in range(4))

    prob_r = prob_s[pl.ds(1 + c, 1), :]
    s_r = jnp.where(prob_r > SCORE_LOW, prob_r, 0.0)
    by1_r, bx1_r, by2_r, bx2_r = _decode(
        sy1_r, sx1_r, sy2_r, sx2_r, dy_r, dx_r, dh_r, dw_r
    )
    ra_r = R_SCALE * (
        jnp.maximum(by2_r - by1_r, 0.0) * jnp.maximum(bx2_r - bx1_r, 0.0)
    )
    i_idx = jax.lax.broadcasted_iota(jnp.int32, (1, N_PAD), 1)

    pack = jnp.concatenate(
        [by1_r, bx1_r, by2_r, bx2_r, ra_r, s_r, ra_r, s_r], axis=0
    )
    packT = pack.T

    suppressed = jnp.zeros((1, N_PAD), jnp.bool_)
    for j0 in range(0, N_PAD, J_CHUNK):
        tp = packT[j0 : j0 + J_CHUNK, :]
        by1_c, bx1_c, by2_c, bx2_c, ra_c, s_c = (
            tp[:, k : k + 1] for k in range(6)
        )
        j_idx = jax.lax.broadcasted_iota(jnp.int32, (J_CHUNK, 1), 0) + j0

        iy1 = jnp.maximum(by1_c, by1_r)
        ix1 = jnp.maximum(bx1_c, bx1_r)
        iy2 = jnp.minimum(by2_c, by2_r)
        ix2 = jnp.minimum(bx2_c, bx2_r)
        inter = jnp.maximum(iy2 - iy1, 0.0) * jnp.maximum(ix2 - ix1, 0.0)
        over = inter > (ra_c + ra_r)
        higher = (s_c > s_r) | ((s_c == s_r) & (j_idx < i_idx))
        supp = jnp.any(over & higher, axis=0, keepdims=True)
        suppressed = suppressed | supp

    keep = jnp.logical_not(suppressed) & (s_r > SCORE_LOW)
    out_s = jnp.where(keep, s_r, 0.0)

    bbox_ref[0] = packT[:N_ROI, 0:4]
    score_ref[0] = out_s[:, :N_ROI]


@jax.jit
def kernel(rois, roi_cls_loc, roi_score):
    f = jnp.float32
    loc3 = roi_cls_loc.reshape(N_ROI, N_CLASS, 4)

    locrow = jnp.full((N_FG, 8, N_PAD), NEG, f)
    locrow = locrow.at[:, 0:4, :N_ROI].set(loc3[:, 1:, :].transpose(1, 2, 0))
    auxrow = jnp.full((32, N_PAD), NEG, f)
    auxrow = auxrow.at[0:4, :N_ROI].set(rois.T)
    auxrow = auxrow.at[4 : 4 + N_CLASS, :N_ROI].set(roi_score.T)

    bboxes, scores = pl.pallas_call(
        _nms_kernel,
        grid=(N_FG,),
        in_specs=[
            pl.BlockSpec((1, 8, N_PAD), lambda c: (c, 0, 0)),
            pl.BlockSpec((32, N_PAD), lambda c: (0, 0)),
        ],
        out_specs=[
            pl.BlockSpec((1, N_ROI, 4), lambda c: (c, 0, 0)),
            pl.BlockSpec((1, 1, N_ROI), lambda c: (c, 0, 0)),
        ],
        out_shape=[
            jax.ShapeDtypeStruct((N_FG, N_ROI, 4), f),
            jax.ShapeDtypeStruct((N_FG, 1, N_ROI), f),
        ],
        scratch_shapes=[pltpu.VMEM((24, N_PAD), f)],
    )(locrow, auxrow)

    return bboxes, scores[:, 0, :]
